# baseline (speedup 1.0000x reference)
import functools

import jax
import jax.numpy as jnp
from jax import lax
from jax.experimental import pallas as pl
from jax.experimental.pallas import tpu as pltpu
from jax.experimental.pallas import tpu_sc as plsc

F = 26
B = 4096
L = 20
V = 100000
D = 4

NC = 2
NS = 16
NW = NC * NS

PV = 102400
BVA = 10240
ROWS = F * PV


BAGS = F * B
BAGS_PER_TILE = BAGS // NW
CHUNK_BAGS = 416
NCHUNKS = BAGS_PER_TILE // CHUNK_BAGS
CHUNK_IDX = CHUNK_BAGS * L


def _proj_tc_body(w_ref, t_ref, p_ref):
    i = pl.program_id(0)
    j = pl.program_id(1)
    acc = lax.dot_general(
        t_ref[0], w_ref[...],
        (((1,), (1,)), ((), ())),
        preferred_element_type=jnp.float32,
    )[:, 0]
    p_ref[pl.ds(i * PV + j * BVA, BVA)] = acc


def _project(tables, W):
    return pl.pallas_call(
        _proj_tc_body,
        grid=(F, PV // BVA),
        in_specs=[
            pl.BlockSpec((1, D), lambda i, j: (0, 0)),
            pl.BlockSpec((1, BVA, D), lambda i, j: (i, j, 0)),
        ],
        out_specs=pl.BlockSpec((ROWS,), lambda i, j: (0,)),
        out_shape=jax.ShapeDtypeStruct((ROWS,), jnp.float32),
    )(W, tables)


BB = 1024


def _pad_tc_body(i_ref, o_ref):
    f = pl.program_id(0)
    o_ref[:, pl.ds(0, L)] = i_ref[0] + f * PV


def _flatten_idx(indices):
    return pl.pallas_call(
        _pad_tc_body,
        grid=(F, B // BB),
        in_specs=[pl.BlockSpec((1, BB, L), lambda i, j: (i, j, 0))],
        out_specs=pl.BlockSpec((BB, 128), lambda i, j: (i * (B // BB) + j, 0)),
        out_shape=jax.ShapeDtypeStruct((BAGS, 128), jnp.int32),
    )(indices).reshape(BAGS * 128)


def _lookup_body(idx_hbm, p_hbm, wb_hbm, out_hbm, ipad_v, cidx_v, vals_v, wb_v, out_v, sem):
    wid = lax.axis_index("s") * NC + lax.axis_index("c")

    pltpu.sync_copy(wb_hbm, wb_v)
    bias = wb_v[0]
    iota16 = lax.iota(jnp.int32, 16)

    def chunk_body(c, carry):
        gbag0 = wid * BAGS_PER_TILE + c * CHUNK_BAGS
        goff = gbag0 * L

        pltpu.sync_copy(idx_hbm.at[pl.ds(gbag0 * 128, CHUNK_BAGS * 128)], ipad_v)

        def compact(g, cr):
            base = (iota16 + g * 16)
            for l in range(L):
                v = plsc.load_gather(ipad_v, [base * 128 + l])
                plsc.store_scatter(cidx_v, [base * L + l], v)
            return cr

        lax.fori_loop(0, CHUNK_BAGS // 16, compact, 0)

        pltpu.async_copy(p_hbm.at[cidx_v], vals_v, sem).wait()

        def pool(g, cr):
            pv = (iota16 + g * 16) * L
            acc = plsc.load_gather(vals_v, [pv])
            for l in range(1, L):
                acc = acc + plsc.load_gather(vals_v, [pv + l])
            out_v[pl.ds(g * 16, 16)] = acc + bias
            return cr

        lax.fori_loop(0, CHUNK_BAGS // 16, pool, 0)

        pltpu.sync_copy(out_v, out_hbm.at[pl.ds(gbag0, CHUNK_BAGS)])
        return carry

    lax.fori_loop(0, NCHUNKS, chunk_body, 0)


_MESH = plsc.VectorSubcoreMesh(
    core_axis_name="c", subcore_axis_name="s", num_cores=NC, num_subcores=NS
)

_lookup_call = functools.partial(
    pl.kernel,
    out_type=jax.ShapeDtypeStruct((BAGS,), jnp.float32),
    mesh=_MESH,
    compiler_params=pltpu.CompilerParams(needs_layout_passes=False),
    scratch_types=[
        pltpu.VMEM((CHUNK_BAGS * 128,), jnp.int32),
        pltpu.VMEM((CHUNK_IDX,), jnp.int32),
        pltpu.VMEM((CHUNK_IDX,), jnp.float32),
        pltpu.VMEM((1, 16), jnp.float32),
        pltpu.VMEM((CHUNK_BAGS,), jnp.float32),
        pltpu.SemaphoreType.DMA,
    ],
)(_lookup_body)


@jax.jit
def kernel(indices, tables, W, b):
    p = _project(tables, W)
    gidx = _flatten_idx(indices)
    bvec = jnp.broadcast_to(b.reshape(1, 1), (1, 16))
    out = _lookup_call(gidx, p, bvec)
    return out.reshape(BAGS, 1)
